# baseline (speedup 1.0000x reference)
import jax
import jax.numpy as jnp
from jax.experimental import pallas as pl
from jax.experimental.pallas import tpu as pltpu

_IN = 2048
_HID = 32768
_K = 64
_B = 128
_BLK = 1024
_NBLK = _HID // _BLK
_CCHUNK = 8192


def _fused_body(x_ref, we_ref, be_ref, wd_ref, xhat_ref, zd_ref, zs_ref,
                zall_ref, thr_ref):
    i = pl.program_id(0)

    @pl.when(i < _NBLK)
    def _encode():
        z = jax.lax.dot_general(
            x_ref[...], we_ref[...], (((1,), (1,)), ((), ())),
            preferred_element_type=jnp.float32)
        z = jnp.maximum(z + be_ref[...], 0.0)
        zd_ref[...] = z
        zall_ref[:, pl.ds(i * _BLK, _BLK)] = z

    @pl.when(i == _NBLK)
    def _find_threshold():
        def cond(state):
            t, _, alldone = state
            return (t < 31) & jnp.logical_not(alldone)

        def step(state):
            t, thr, _ = state
            bit = 30 - t
            done = thr < 0
            cand = (thr & 0x7FFFFFFF) | (1 << bit)
            cand_f = jax.lax.bitcast_convert_type(cand, jnp.float32)
            cnt = jnp.zeros((_B, 1), jnp.float32)
            for c in range(_HID // _CCHUNK):
                zc = zall_ref[:, c * _CCHUNK:(c + 1) * _CCHUNK]
                cnt = cnt + jnp.sum((zc >= cand_f).astype(jnp.float32),
                                    axis=1, keepdims=True)
            take = jnp.logical_and(cnt >= float(_K), jnp.logical_not(done))
            thr = jnp.where(take, cand, thr)
            newly = jnp.logical_and(take, cnt == float(_K))
            thr = jnp.where(newly, thr | jnp.int32(-2147483648), thr)
            alldone = jnp.all(thr < 0)
            return (t + 1, thr, alldone)

        _, thr, _ = jax.lax.while_loop(
            cond, step, (0, jnp.zeros((_B, 1), jnp.int32), False))
        thr_ref[...] = jax.lax.bitcast_convert_type(
            thr & 0x7FFFFFFF, jnp.float32)

    @pl.when(i >= _NBLK)
    def _decode():
        j = i - _NBLK
        zb = zall_ref[:, pl.ds(j * _BLK, _BLK)]
        zs = jnp.where(zb >= thr_ref[...], zb, 0.0)
        zs_ref[...] = zs
        part = jax.lax.dot_general(
            zs, wd_ref[...], (((1,), (1,)), ((), ())),
            preferred_element_type=jnp.float32)

        @pl.when(i == _NBLK)
        def _init():
            xhat_ref[...] = part

        @pl.when(i > _NBLK)
        def _acc():
            xhat_ref[...] += part


def kernel(x, W_enc, b_enc, W_dec):
    x_hat, z_dense, z_sparse = pl.pallas_call(
        _fused_body,
        grid=(2 * _NBLK,),
        in_specs=[
            pl.BlockSpec((_B, _IN), lambda i: (0, 0)),
            pl.BlockSpec((_BLK, _IN), lambda i: (jnp.minimum(i, _NBLK - 1), 0)),
            pl.BlockSpec((1, _BLK), lambda i: (0, jnp.minimum(i, _NBLK - 1))),
            pl.BlockSpec((_IN, _BLK),
                         lambda i: (0, jnp.maximum(i - _NBLK, 0))),
        ],
        out_specs=[
            pl.BlockSpec((_B, _IN), lambda i: (0, 0)),
            pl.BlockSpec((_B, _BLK), lambda i: (0, jnp.minimum(i, _NBLK - 1))),
            pl.BlockSpec((_B, _BLK),
                         lambda i: (0, jnp.maximum(i - _NBLK, 0))),
        ],
        out_shape=[
            jax.ShapeDtypeStruct((_B, _IN), jnp.float32),
            jax.ShapeDtypeStruct((_B, _HID), jnp.float32),
            jax.ShapeDtypeStruct((_B, _HID), jnp.float32),
        ],
        scratch_shapes=[
            pltpu.VMEM((_B, _HID), jnp.float32),
            pltpu.VMEM((_B, 1), jnp.float32),
        ],
    )(x, W_enc, b_enc.reshape(1, _HID), W_dec)

    return (x_hat, z_dense, z_sparse)


# speedup vs baseline: 1.0115x; 1.0115x over previous
import jax
import jax.numpy as jnp
from jax.experimental import pallas as pl
from jax.experimental.pallas import tpu as pltpu

_IN = 2048
_HID = 32768
_K = 64
_B = 128
_BLK = 1024
_NBLK = _HID // _BLK
_CCHUNK = 8192


def _wd_dma(wd_ref, wdb_ref, sem_ref, blk, slot):
    return pltpu.make_async_copy(
        wd_ref.at[:, pl.ds(blk * _BLK, _BLK)], wdb_ref.at[slot],
        sem_ref.at[slot])


def _fused_body(x_ref, we_ref, be_ref, wd_ref, xhat_ref, zd_ref, zs_ref,
                zall_ref, thr_ref, wdb_ref, sem_ref):
    i = pl.program_id(0)

    @pl.when(i < _NBLK)
    def _encode():
        z = jax.lax.dot_general(
            x_ref[...], we_ref[...], (((1,), (1,)), ((), ())),
            preferred_element_type=jnp.float32)
        z = jnp.maximum(z + be_ref[...], 0.0)
        zd_ref[...] = z
        zall_ref[:, pl.ds(i * _BLK, _BLK)] = z

    @pl.when(i == _NBLK)
    def _prime_ring():
        for s in range(2):
            _wd_dma(wd_ref, wdb_ref, sem_ref, s, s).start()

    @pl.when(i == _NBLK)
    def _find_threshold():
        def cond(state):
            t, _, alldone = state
            return (t < 31) & jnp.logical_not(alldone)

        def step(state):
            t, thr, _ = state
            bit = 30 - t
            done = thr < 0
            cand = (thr & 0x7FFFFFFF) | (1 << bit)
            cand_f = jax.lax.bitcast_convert_type(cand, jnp.float32)
            cnt = jnp.zeros((_B, 1), jnp.float32)
            for c in range(_HID // _CCHUNK):
                zc = zall_ref[:, c * _CCHUNK:(c + 1) * _CCHUNK]
                cnt = cnt + jnp.sum((zc >= cand_f).astype(jnp.float32),
                                    axis=1, keepdims=True)
            take = jnp.logical_and(cnt >= float(_K), jnp.logical_not(done))
            thr = jnp.where(take, cand, thr)
            newly = jnp.logical_and(take, cnt == float(_K))
            thr = jnp.where(newly, thr | jnp.int32(-2147483648), thr)
            alldone = jnp.all(thr < 0)
            return (t + 1, thr, alldone)

        _, thr, _ = jax.lax.while_loop(
            cond, step, (0, jnp.zeros((_B, 1), jnp.int32), False))
        thr_ref[...] = jax.lax.bitcast_convert_type(
            thr & 0x7FFFFFFF, jnp.float32)

    @pl.when(i >= _NBLK)
    def _decode():
        j = i - _NBLK
        slot = jax.lax.rem(j, 2)
        _wd_dma(wd_ref, wdb_ref, sem_ref, j, slot).wait()
        zb = zall_ref[:, pl.ds(j * _BLK, _BLK)]
        zs = jnp.where(zb >= thr_ref[...], zb, 0.0)
        zs_ref[...] = zs
        part = jax.lax.dot_general(
            zs, wdb_ref[slot], (((1,), (1,)), ((), ())),
            preferred_element_type=jnp.float32)

        @pl.when(i == _NBLK)
        def _init():
            xhat_ref[...] = part

        @pl.when(i > _NBLK)
        def _acc():
            xhat_ref[...] += part

        @pl.when(j < _NBLK - 2)
        def _refill():
            _wd_dma(wd_ref, wdb_ref, sem_ref, j + 2, slot).start()


def kernel(x, W_enc, b_enc, W_dec):
    x_hat, z_dense, z_sparse = pl.pallas_call(
        _fused_body,
        grid=(2 * _NBLK,),
        in_specs=[
            pl.BlockSpec((_B, _IN), lambda i: (0, 0)),
            pl.BlockSpec((_BLK, _IN), lambda i: (jnp.minimum(i, _NBLK - 1), 0)),
            pl.BlockSpec((1, _BLK), lambda i: (0, jnp.minimum(i, _NBLK - 1))),
            pl.BlockSpec(memory_space=pl.ANY),
        ],
        out_specs=[
            pl.BlockSpec((_B, _IN), lambda i: (0, 0)),
            pl.BlockSpec((_B, _BLK), lambda i: (0, jnp.minimum(i, _NBLK - 1))),
            pl.BlockSpec((_B, _BLK),
                         lambda i: (0, jnp.maximum(i - _NBLK, 0))),
        ],
        out_shape=[
            jax.ShapeDtypeStruct((_B, _IN), jnp.float32),
            jax.ShapeDtypeStruct((_B, _HID), jnp.float32),
            jax.ShapeDtypeStruct((_B, _HID), jnp.float32),
        ],
        scratch_shapes=[
            pltpu.VMEM((_B, _HID), jnp.float32),
            pltpu.VMEM((_B, 1), jnp.float32),
            pltpu.VMEM((2, _IN, _BLK), jnp.float32),
            pltpu.SemaphoreType.DMA((2,)),
        ],
    )(x, W_enc, b_enc.reshape(1, _HID), W_dec)

    return (x_hat, z_dense, z_sparse)
